# Initial kernel scaffold; baseline (speedup 1.0000x reference)
import functools

import jax
import jax.numpy as jnp
from jax import lax
from jax.experimental import pallas as pl
from jax.experimental.pallas import tpu as pltpu
from jax.experimental.pallas import tpu_sc as plsc

N = 10000
E = 320000
D = 128
NC = 2
NS = 16
NW = NC * NS
CHUNK = 128
C_PER_W = (E + NW * CHUNK - 1) // (NW * CHUNK)
E_PAD = NW * C_PER_W * CHUNK
N_PAD = 10240
ROWS_PER_TILE = N_PAD // NS
DW = 16

_mesh = plsc.VectorSubcoreMesh(core_axis_name="c", subcore_axis_name="s")


def _deg_body(src3, dst3, ones_hbm, zeros_hbm, degp, sidx, didx, ones_v,
              acc_s, acc_d):
    c = lax.axis_index("c")
    s = lax.axis_index("s")
    w = c * NS + s
    row0 = s * ROWS_PER_TILE
    pltpu.sync_copy(zeros_hbm, acc_s.at[pl.ds(row0, ROWS_PER_TILE)])
    pltpu.sync_copy(zeros_hbm, acc_d.at[pl.ds(row0, ROWS_PER_TILE)])
    pltpu.sync_copy(ones_hbm, ones_v)
    plsc.subcore_barrier()

    def body(j, carry):
        pltpu.sync_copy(src3.at[w, j], sidx)
        pltpu.sync_copy(dst3.at[w, j], didx)
        pltpu.sync_copy(ones_v, acc_s.at[sidx], add=True)
        pltpu.sync_copy(ones_v, acc_d.at[didx], add=True)
        return carry

    lax.fori_loop(0, C_PER_W, body, 0)
    plsc.subcore_barrier()
    sl = pl.ds(row0, ROWS_PER_TILE)
    pltpu.sync_copy(acc_s.at[sl], degp.at[0, c, sl])
    pltpu.sync_copy(acc_d.at[sl], degp.at[1, c, sl])


_deg_kernel = pl.kernel(
    _deg_body,
    out_type=jax.ShapeDtypeStruct((2, NC, N_PAD, DW), jnp.float32),
    mesh=_mesh,
    scratch_types=[
        pltpu.VMEM((CHUNK,), jnp.int32),
        pltpu.VMEM((CHUNK,), jnp.int32),
        pltpu.VMEM((CHUNK, DW), jnp.float32),
        pltpu.VMEM_SHARED((N_PAD, DW), jnp.float32),
        pltpu.VMEM_SHARED((N_PAD, DW), jnp.float32),
    ],
)


def _edge_body(m_hbm, src3, dst3, zeros_hbm, part, sidx, didx, rows, acc):
    c = lax.axis_index("c")
    s = lax.axis_index("s")
    w = c * NS + s
    row0 = s * ROWS_PER_TILE
    pltpu.sync_copy(zeros_hbm, acc.at[pl.ds(row0, ROWS_PER_TILE)])
    plsc.subcore_barrier()

    def body(j, carry):
        pltpu.sync_copy(src3.at[w, j], sidx)
        pltpu.sync_copy(dst3.at[w, j], didx)
        pltpu.sync_copy(m_hbm.at[sidx], rows)
        pltpu.sync_copy(rows, acc.at[didx], add=True)
        return carry

    lax.fori_loop(0, C_PER_W, body, 0)
    plsc.subcore_barrier()
    sl = pl.ds(row0, ROWS_PER_TILE)
    pltpu.sync_copy(acc.at[sl], part.at[c, sl])


_edge_kernel = pl.kernel(
    _edge_body,
    out_type=jax.ShapeDtypeStruct((NC, N_PAD, D), jnp.float32),
    mesh=_mesh,
    scratch_types=[
        pltpu.VMEM((CHUNK,), jnp.int32),
        pltpu.VMEM((CHUNK,), jnp.int32),
        pltpu.VMEM((CHUNK, D), jnp.float32),
        pltpu.VMEM_SHARED((N_PAD, D), jnp.float32),
    ],
)


def _norm_body(degp_ref, x_ref, onorm_ref, inorm_ref, m0_ref):
    ds_ = degp_ref[0, 0, :, 0:1] + degp_ref[0, 1, :, 0:1]
    dd_ = degp_ref[1, 0, :, 0:1] + degp_ref[1, 1, :, 0:1]
    mask = (lax.broadcasted_iota(jnp.int32, (N_PAD, 1), 0) < N).astype(
        jnp.float32)
    onorm = lax.rsqrt(jnp.maximum(ds_, 1.0)) * mask
    inorm = lax.rsqrt(jnp.maximum(dd_, 1.0))
    onorm_ref[...] = onorm
    inorm_ref[...] = inorm
    m0_ref[...] = x_ref[...] * onorm


def _norm_kernel(degp, x_pad):
    return pl.pallas_call(
        _norm_body,
        out_shape=(
            jax.ShapeDtypeStruct((N_PAD, 1), jnp.float32),
            jax.ShapeDtypeStruct((N_PAD, 1), jnp.float32),
            jax.ShapeDtypeStruct((N_PAD, D), jnp.float32),
        ),
    )(degp, x_pad)


def _layer_body(p_ref, inorm_ref, onorm_ref, w_ref, b_ref, mnext_ref):
    agg = (p_ref[0] + p_ref[1]) * inorm_ref[...]
    z = jnp.dot(agg, w_ref[...], preferred_element_type=jnp.float32)
    h = jnp.maximum(z + b_ref[...], 0.0)
    mnext_ref[...] = h * onorm_ref[...]


def _layer_kernel(part, inorm, onorm, w, b):
    return pl.pallas_call(
        _layer_body,
        out_shape=jax.ShapeDtypeStruct((N_PAD, D), jnp.float32),
    )(part, inorm, onorm, w, b)


def _final_body(p_ref, inorm_ref, w_ref, b_ref, wc_ref, bc_ref, out_ref):
    agg = (p_ref[0] + p_ref[1]) * inorm_ref[...]
    z = jnp.dot(agg, w_ref[...], preferred_element_type=jnp.float32)
    mask = (lax.broadcasted_iota(jnp.int32, (N_PAD, 1), 0) < N).astype(
        jnp.float32)
    h = jnp.maximum(z + b_ref[...], 0.0) * mask
    hg = jnp.sum(h, axis=0, keepdims=True) * (1.0 / N)
    out_ref[...] = (
        jnp.dot(hg, wc_ref[...], preferred_element_type=jnp.float32)
        + bc_ref[...])


def _final_kernel(part, inorm, w, b, wc, bc):
    return pl.pallas_call(
        _final_body,
        out_shape=jax.ShapeDtypeStruct((1, 2), jnp.float32),
    )(part, inorm, w, b, wc, bc)


@jax.jit
def kernel(x, edge_index, W0, b0, W1, b1, W2, b2, Wc, bc):
    pad = jnp.full((E_PAD - E,), N, dtype=jnp.int32)
    src3 = jnp.concatenate([edge_index[0], pad]).reshape(NW, C_PER_W, CHUNK)
    dst3 = jnp.concatenate([edge_index[1], pad]).reshape(NW, C_PER_W, CHUNK)
    x_pad = jnp.concatenate(
        [x, jnp.zeros((N_PAD - N, D), jnp.float32)], axis=0)
    ones_rows = jnp.ones((CHUNK, DW), jnp.float32)
    zeros_deg = jnp.zeros((ROWS_PER_TILE, DW), jnp.float32)
    zeros_row = jnp.zeros((ROWS_PER_TILE, D), jnp.float32)

    degp = _deg_kernel(src3, dst3, ones_rows, zeros_deg)
    onorm, inorm, m = _norm_kernel(degp, x_pad)

    for (w, b) in ((W0, b0), (W1, b1)):
        part = _edge_kernel(m, src3, dst3, zeros_row)
        m = _layer_kernel(part, inorm, onorm, w, b.reshape(1, D))

    part = _edge_kernel(m, src3, dst3, zeros_row)
    return _final_kernel(part, inorm, W2, b2.reshape(1, D), Wc,
                         bc.reshape(1, 2))


# speedup vs baseline: 3.4984x; 3.4984x over previous
import functools

import jax
import jax.numpy as jnp
from jax import lax
from jax.experimental import pallas as pl
from jax.experimental.pallas import tpu as pltpu
from jax.experimental.pallas import tpu_sc as plsc

N = 10000
E = 320000
D = 128
NC = 2
NS = 16
NW = NC * NS
CHUNK = 128
C_PER_W = (E + NW * CHUNK - 1) // (NW * CHUNK)
E_PAD = NW * C_PER_W * CHUNK
N_CHUNKS = E_PAD // CHUNK
C_PER_TILE = N_CHUNKS // NS
N_PAD = 10240
ROWS_PER_TILE = N_PAD // NS

_mesh = plsc.VectorSubcoreMesh(core_axis_name="c", subcore_axis_name="s")


def _deg_body(idxs, ones_hbm, zeros_hbm, degp, sidx, ones_v, acc):
    c = lax.axis_index("c")
    s = lax.axis_index("s")
    row0 = s * ROWS_PER_TILE
    pltpu.sync_copy(zeros_hbm, acc.at[pl.ds(row0, ROWS_PER_TILE)])
    pltpu.sync_copy(ones_hbm, ones_v)
    plsc.subcore_barrier()

    def body(j, carry):
        pltpu.sync_copy(idxs.at[c, s * C_PER_TILE + j], sidx)
        pltpu.sync_copy(ones_v, acc.at[sidx], add=True)
        return carry

    lax.fori_loop(0, C_PER_TILE, body, 0)
    plsc.subcore_barrier()
    sl = pl.ds(row0, ROWS_PER_TILE)
    pltpu.sync_copy(acc.at[sl], degp.at[c, sl])


_deg_kernel = pl.kernel(
    _deg_body,
    out_type=jax.ShapeDtypeStruct((NC, N_PAD, D), jnp.float32),
    mesh=_mesh,
    scratch_types=[
        pltpu.VMEM((CHUNK,), jnp.int32),
        pltpu.VMEM((CHUNK, D), jnp.float32),
        pltpu.VMEM_SHARED((N_PAD, D), jnp.float32),
    ],
)


def _edge_body(m_hbm, src3, dst3, zeros_hbm, part, sidx, didx, rows, acc):
    c = lax.axis_index("c")
    s = lax.axis_index("s")
    w = c * NS + s
    row0 = s * ROWS_PER_TILE
    pltpu.sync_copy(zeros_hbm, acc.at[pl.ds(row0, ROWS_PER_TILE)])
    plsc.subcore_barrier()

    def body(j, carry):
        pltpu.sync_copy(src3.at[w, j], sidx)
        pltpu.sync_copy(dst3.at[w, j], didx)
        pltpu.sync_copy(m_hbm.at[sidx], rows)
        pltpu.sync_copy(rows, acc.at[didx], add=True)
        return carry

    lax.fori_loop(0, C_PER_W, body, 0)
    plsc.subcore_barrier()
    sl = pl.ds(row0, ROWS_PER_TILE)
    pltpu.sync_copy(acc.at[sl], part.at[c, sl])


_edge_kernel = pl.kernel(
    _edge_body,
    out_type=jax.ShapeDtypeStruct((NC, N_PAD, D), jnp.float32),
    mesh=_mesh,
    scratch_types=[
        pltpu.VMEM((CHUNK,), jnp.int32),
        pltpu.VMEM((CHUNK,), jnp.int32),
        pltpu.VMEM((CHUNK, D), jnp.float32),
        pltpu.VMEM_SHARED((N_PAD, D), jnp.float32),
    ],
)


def _norm_body(degp_ref, x_ref, onorm_ref, inorm_ref, m0_ref):
    ds_ = degp_ref[0, :, 0:1]
    dd_ = degp_ref[1, :, 0:1]
    mask = (lax.broadcasted_iota(jnp.int32, (N_PAD, 1), 0) < N).astype(
        jnp.float32)
    onorm = lax.rsqrt(jnp.maximum(ds_, 1.0)) * mask
    inorm = lax.rsqrt(jnp.maximum(dd_, 1.0))
    onorm_ref[...] = onorm
    inorm_ref[...] = inorm
    m0_ref[...] = x_ref[...] * onorm


def _norm_kernel(degp, x_pad):
    return pl.pallas_call(
        _norm_body,
        out_shape=(
            jax.ShapeDtypeStruct((N_PAD, 1), jnp.float32),
            jax.ShapeDtypeStruct((N_PAD, 1), jnp.float32),
            jax.ShapeDtypeStruct((N_PAD, D), jnp.float32),
        ),
    )(degp, x_pad)


def _layer_body(p_ref, inorm_ref, onorm_ref, w_ref, b_ref, mnext_ref):
    agg = (p_ref[0] + p_ref[1]) * inorm_ref[...]
    z = jnp.dot(agg, w_ref[...], preferred_element_type=jnp.float32)
    h = jnp.maximum(z + b_ref[...], 0.0)
    mnext_ref[...] = h * onorm_ref[...]


def _layer_kernel(part, inorm, onorm, w, b):
    return pl.pallas_call(
        _layer_body,
        out_shape=jax.ShapeDtypeStruct((N_PAD, D), jnp.float32),
    )(part, inorm, onorm, w, b)


def _final_body(p_ref, inorm_ref, w_ref, b_ref, wc_ref, bc_ref, out_ref):
    agg = (p_ref[0] + p_ref[1]) * inorm_ref[...]
    z = jnp.dot(agg, w_ref[...], preferred_element_type=jnp.float32)
    mask = (lax.broadcasted_iota(jnp.int32, (N_PAD, 1), 0) < N).astype(
        jnp.float32)
    h = jnp.maximum(z + b_ref[...], 0.0) * mask
    hg = jnp.sum(h, axis=0, keepdims=True) * (1.0 / N)
    out_ref[...] = (
        jnp.dot(hg, wc_ref[...], preferred_element_type=jnp.float32)
        + bc_ref[...])


def _final_kernel(part, inorm, w, b, wc, bc):
    return pl.pallas_call(
        _final_body,
        out_shape=jax.ShapeDtypeStruct((1, 2), jnp.float32),
    )(part, inorm, w, b, wc, bc)


@jax.jit
def kernel(x, edge_index, W0, b0, W1, b1, W2, b2, Wc, bc):
    pad = jnp.full((E_PAD - E,), N, dtype=jnp.int32)
    src3 = jnp.concatenate([edge_index[0], pad]).reshape(NW, C_PER_W, CHUNK)
    dst3 = jnp.concatenate([edge_index[1], pad]).reshape(NW, C_PER_W, CHUNK)
    x_pad = jnp.concatenate(
        [x, jnp.zeros((N_PAD - N, D), jnp.float32)], axis=0)
    idxs = jnp.stack([src3.reshape(N_CHUNKS, CHUNK),
                      dst3.reshape(N_CHUNKS, CHUNK)])
    ones_rows = jnp.ones((CHUNK, D), jnp.float32)
    zeros_row = jnp.zeros((ROWS_PER_TILE, D), jnp.float32)

    degp = _deg_kernel(idxs, ones_rows, zeros_row)
    onorm, inorm, m = _norm_kernel(degp, x_pad)

    for (w, b) in ((W0, b0), (W1, b1)):
        part = _edge_kernel(m, src3, dst3, zeros_row)
        m = _layer_kernel(part, inorm, onorm, w, b.reshape(1, D))

    part = _edge_kernel(m, src3, dst3, zeros_row)
    return _final_kernel(part, inorm, W2, b2.reshape(1, D), Wc,
                         bc.reshape(1, 2))
